# baseline (speedup 1.0000x reference)
import functools

import jax
import jax.numpy as jnp
from jax import lax
from jax.experimental import pallas as pl
from jax.experimental.pallas import tpu as pltpu
from jax.experimental.pallas import tpu_sc as plsc

_TG = 256
_TT = 512


def _router_body(nt, e_count, t_total, tg, n_tiles,
                 xb_ref, rw_ref, rb_ref,
                 ep_ref, wp_ref, rk_ref, offs_ref, te_ref, aux_ref,
                 carry_ref, fsum_ref, psum_ref):
    s = pl.program_id(0)
    t = pl.program_id(1)
    first = (s == 0) & (t == 0)

    @pl.when(first)
    def _():
        carry_ref[...] = jnp.zeros_like(carry_ref)
        fsum_ref[...] = jnp.zeros_like(fsum_ref)
        psum_ref[...] = jnp.zeros_like(psum_ref)

    xt = xb_ref[...]
    logits = lax.dot_general(
        xt, rw_ref[...], (((1,), (1,)), ((), ())),
        precision=lax.Precision.DEFAULT,
        preferred_element_type=jnp.float32) + rb_ref[...]
    tt, ee = logits.shape
    ids = lax.broadcasted_iota(jnp.int32, (tt, ee), 1)
    m1 = jnp.max(logits, axis=1, keepdims=True)
    a1 = jnp.min(jnp.where(logits == m1, ids, ee), axis=1, keepdims=True)
    sel1 = ids == a1
    rest = jnp.where(sel1, -jnp.inf, logits)
    m2 = jnp.max(rest, axis=1, keepdims=True)
    a2 = jnp.min(jnp.where(rest == m2, ids, ee), axis=1, keepdims=True)
    sel2 = ids == a2
    ez = jnp.where(sel1 | sel2, jnp.exp(logits - m1), 0.0)
    w = ez / jnp.sum(ez, axis=1, keepdims=True)

    @pl.when(s == 0)
    def _():
        fsum_ref[...] += jnp.sum((sel1 | sel2).astype(jnp.float32),
                                 axis=0)[None, :]
        psum_ref[...] += jnp.sum(w, axis=0)[None, :]

    is0 = (s == 0)
    sel = jnp.where(is0, sel1.astype(jnp.float32), sel2.astype(jnp.float32))
    a_slot = jnp.where(is0, a1, a2)
    ep_ref[...] = a_slot
    wp_ref[...] = jnp.sum(sel * w, axis=1, keepdims=True)

    ri = lax.broadcasted_iota(jnp.int32, (tt, tt), 0)
    ci = lax.broadcasted_iota(jnp.int32, (tt, tt), 1)
    tril = (ci < ri).astype(jnp.float32)
    excl = lax.dot_general(
        tril, sel, (((1,), (0,)), ((), ())),
        precision=lax.Precision.HIGHEST,
        preferred_element_type=jnp.float32)
    rank_all = excl + carry_ref[...]
    rank_pair = jnp.sum(rank_all * sel, axis=1, keepdims=True)
    rk_ref[...] = (rank_pair + 0.5).astype(jnp.int32)
    carry_ref[...] += jnp.sum(sel, axis=0)[None, :]

    @pl.when((s == 1) & (t == nt - 1))
    def _fin():
        counts = carry_ref[...]
        pc = jnp.floor((counts + (tg - 1)) * (1.0 / tg)) * tg
        rj = lax.broadcasted_iota(jnp.int32, (ee, 16), 0)
        cj = lax.broadcasted_iota(jnp.int32, (ee, 16), 1)
        upmask = (rj < cj).astype(jnp.float32)
        offs = lax.dot_general(
            pc, upmask, (((1,), (0,)), ((), ())),
            precision=lax.Precision.HIGHEST,
            preferred_element_type=jnp.float32)
        offs_ref[...] = (offs + 0.5).astype(jnp.int32)
        ends = offs[:, :ee] + pc
        tvec = lax.broadcasted_iota(
            jnp.int32, (1, n_tiles), 1).astype(jnp.float32) * tg
        te = jnp.zeros((1, n_tiles), jnp.float32)
        for e in range(e_count):
            te += (ends[:, e:e + 1] <= tvec).astype(jnp.float32)
        te = jnp.minimum(te, e_count - 1)
        te_ref[...] = (te + 0.5).astype(jnp.int32)
        f = fsum_ref[...] * (1.0 / t_total)
        p = psum_ref[...] * (1.0 / t_total)
        aux_ref[...] = e_count * jnp.sum(f * p, keepdims=True)


def _router(xb, router_w, router_b, E, TG, n_tiles):
    T, D = xb.shape
    nt = T // _TT
    grid = (2, nt)
    return pl.pallas_call(
        functools.partial(_router_body, nt, E, float(T), float(TG), n_tiles),
        grid=grid,
        in_specs=[
            pl.BlockSpec((_TT, D), lambda s, t: (t, 0)),
            pl.BlockSpec((E, D), lambda s, t: (0, 0)),
            pl.BlockSpec((1, E), lambda s, t: (0, 0)),
        ],
        out_specs=[
            pl.BlockSpec((_TT, 1), lambda s, t: (s * nt + t, 0)),
            pl.BlockSpec((_TT, 1), lambda s, t: (s * nt + t, 0)),
            pl.BlockSpec((_TT, 1), lambda s, t: (s * nt + t, 0)),
            pl.BlockSpec((1, 16), lambda s, t: (0, 0)),
            pl.BlockSpec((1, n_tiles), lambda s, t: (0, 0)),
            pl.BlockSpec((1, 1), lambda s, t: (0, 0)),
        ],
        out_shape=[
            jax.ShapeDtypeStruct((2 * T, 1), jnp.int32),
            jax.ShapeDtypeStruct((2 * T, 1), jnp.float32),
            jax.ShapeDtypeStruct((2 * T, 1), jnp.int32),
            jax.ShapeDtypeStruct((1, 16), jnp.int32),
            jax.ShapeDtypeStruct((1, n_tiles), jnp.int32),
            jax.ShapeDtypeStruct((1, 1), jnp.float32),
        ],
        scratch_shapes=[pltpu.VMEM((1, E), jnp.float32),
                        pltpu.VMEM((1, E), jnp.float32),
                        pltpu.VMEM((1, E), jnp.float32)],
    )(xb, router_w, router_b.reshape(1, E))


def _dispatch(ep, rk, wp, offs, P, T, L):
    mesh = plsc.VectorSubcoreMesh(core_axis_name="c", subcore_axis_name="s")
    cpw = P // 32

    @functools.partial(
        pl.kernel, mesh=mesh,
        out_type=[jax.ShapeDtypeStruct((L,), jnp.int32),
                  jax.ShapeDtypeStruct((L,), jnp.float32),
                  jax.ShapeDtypeStruct((P,), jnp.int32)],
        scratch_types=[pltpu.VMEM((cpw,), jnp.int32),
                       pltpu.VMEM((cpw,), jnp.int32),
                       pltpu.VMEM((cpw,), jnp.float32),
                       pltpu.VMEM((16,), jnp.int32),
                       pltpu.VMEM((2, 128), jnp.int32),
                       pltpu.VMEM((2, 128), jnp.int32),
                       pltpu.VMEM((2, 128), jnp.float32),
                       pltpu.SemaphoreType.DMA],
    )
    def disp(ep_hbm, rk_hbm, wp_hbm, offs_hbm,
             stok_hbm, sw_hbm, dest_hbm,
             ep_v, rk_v, wp_v, offs_v, dest_v, tok_v, wsc_v, sem):
        wid = lax.axis_index("s") * 2 + lax.axis_index("c")
        base = wid * cpw
        tokbase = (wid % 16) * cpw
        pltpu.sync_copy(ep_hbm.at[pl.ds(base, cpw)], ep_v)
        pltpu.sync_copy(rk_hbm.at[pl.ds(base, cpw)], rk_v)
        pltpu.sync_copy(wp_hbm.at[pl.ds(base, cpw)], wp_v)
        pltpu.sync_copy(offs_hbm, offs_v)
        offs_vec = offs_v[...]
        dnums = lax.GatherDimensionNumbers(
            offset_dims=(), collapsed_slice_dims=(0,), start_index_map=(0,))
        for h in range(2):
            for i in range(8):
                j = h * 8 + i
                ev = ep_v[pl.ds(j * 16, 16)]
                rv = rk_v[pl.ds(j * 16, 16)]
                og = lax.gather(
                    offs_vec, ev[:, None], dimension_numbers=dnums,
                    slice_sizes=(1,),
                    mode=lax.GatherScatterMode.PROMISE_IN_BOUNDS)
                dest_v[h, pl.ds(i * 16, 16)] = og + rv
                tok_v[h, pl.ds(i * 16, 16)] = (
                    lax.iota(jnp.int32, 16) + (tokbase + j * 16))
                wsc_v[h, pl.ds(i * 16, 16)] = wp_v[pl.ds(j * 16, 16)]
        cp1 = pltpu.async_copy(tok_v.at[0], stok_hbm.at[dest_v.at[0]], sem)
        cp1.wait()
        cp2 = pltpu.async_copy(tok_v.at[1], stok_hbm.at[dest_v.at[1]], sem)
        cp2.wait()
        cp3 = pltpu.async_copy(wsc_v.at[0], sw_hbm.at[dest_v.at[0]], sem)
        cp3.wait()
        cp4 = pltpu.async_copy(wsc_v.at[1], sw_hbm.at[dest_v.at[1]], sem)
        cp4.wait()
        for h in range(2):
            pltpu.sync_copy(dest_v.at[h],
                            dest_hbm.at[pl.ds(base + h * 128, 128)])

    return disp(ep, rk, wp, offs)


def _gather_rows(stok, xi, L, T, Dw):
    mesh = plsc.VectorSubcoreMesh(core_axis_name="c", subcore_axis_name="s")
    spw = L // 32
    CH = 32

    @functools.partial(
        pl.kernel, mesh=mesh,
        out_type=jax.ShapeDtypeStruct((L, Dw), jnp.int32),
        scratch_types=[pltpu.VMEM((CH,), jnp.int32),
                       pltpu.VMEM((CH, Dw), jnp.int32),
                       pltpu.SemaphoreType.DMA],
    )
    def gat(stok_hbm, xi_hbm, xg_hbm, idx_v, rows_v, sem):
        wid = lax.axis_index("s") * 2 + lax.axis_index("c")
        base = wid * spw
        for j in range(spw // CH):
            r0 = base + j * CH
            pltpu.sync_copy(stok_hbm.at[pl.ds(r0, CH)], idx_v)
            for q in range(CH // 16):
                v = idx_v[pl.ds(q * 16, 16)]
                v = jnp.minimum(jnp.maximum(v, 0), T - 1)
                idx_v[pl.ds(q * 16, 16)] = v
            pltpu.async_copy(xi_hbm.at[idx_v], rows_v, sem).wait()
            pltpu.sync_copy(rows_v, xg_hbm.at[pl.ds(r0, CH)])

    return gat(stok, xi)


def _ffn_body(te_ref, xg_ref, wu_ref, bu_ref, wd_ref, bd_ref, w_ref, y_ref):
    h = jnp.dot(xg_ref[...], wu_ref[0], preferred_element_type=jnp.float32)
    h = h + bu_ref[0]
    g = 0.5 * h * (1.0 + lax.erf(h * 0.7071067811865476))
    y = jnp.dot(g.astype(jnp.bfloat16), wd_ref[0],
                preferred_element_type=jnp.float32)
    y_ref[...] = (y + bd_ref[0]) * w_ref[...]


def _ffn(te, xg, wub, bu3, wdb, bd3, sw, L, D, H, E, n_tiles):
    grid_spec = pltpu.PrefetchScalarGridSpec(
        num_scalar_prefetch=1,
        grid=(n_tiles,),
        in_specs=[
            pl.BlockSpec((_TG, D), lambda t, te: (t, 0)),
            pl.BlockSpec((1, D, H), lambda t, te: (te[t], 0, 0)),
            pl.BlockSpec((1, 1, H), lambda t, te: (te[t], 0, 0)),
            pl.BlockSpec((1, H, D), lambda t, te: (te[t], 0, 0)),
            pl.BlockSpec((1, 1, D), lambda t, te: (te[t], 0, 0)),
            pl.BlockSpec((_TG, 1), lambda t, te: (t, 0)),
        ],
        out_specs=pl.BlockSpec((_TG, D), lambda t, te: (t, 0)),
    )
    return pl.pallas_call(
        _ffn_body,
        grid_spec=grid_spec,
        out_shape=jax.ShapeDtypeStruct((L, D), jnp.float32),
    )(te, xg, wub, bu3, wdb, bd3, sw)


def _combine(dest, y, T, L, D):
    mesh = plsc.VectorSubcoreMesh(core_axis_name="c", subcore_axis_name="s")
    tpw = T // 32
    CH = 16

    @functools.partial(
        pl.kernel, mesh=mesh,
        out_type=jax.ShapeDtypeStruct((T, D), jnp.float32),
        scratch_types=[pltpu.VMEM((CH,), jnp.int32),
                       pltpu.VMEM((CH,), jnp.int32),
                       pltpu.VMEM((CH, D), jnp.float32),
                       pltpu.VMEM((CH, D), jnp.float32),
                       pltpu.SemaphoreType.DMA],
    )
    def comb(dest_hbm, y_hbm, out_hbm, iA_v, iB_v, rA_v, rB_v, sem):
        wid = lax.axis_index("s") * 2 + lax.axis_index("c")
        base = wid * tpw
        for j in range(tpw // CH):
            t0 = base + j * CH
            pltpu.sync_copy(dest_hbm.at[pl.ds(t0, CH)], iA_v)
            pltpu.sync_copy(dest_hbm.at[pl.ds(T + t0, CH)], iB_v)
            iA_v[...] = jnp.minimum(jnp.maximum(iA_v[...], 0), L - 1)
            iB_v[...] = jnp.minimum(jnp.maximum(iB_v[...], 0), L - 1)
            pltpu.async_copy(y_hbm.at[iA_v], rA_v, sem).wait()
            pltpu.async_copy(y_hbm.at[iB_v], rB_v, sem).wait()

            ng = D // 16

            def body(i, _):
                r = i // ng
                c = (i % ng) * 16
                rA_v[r, pl.ds(c, 16)] = (rA_v[r, pl.ds(c, 16)]
                                         + rB_v[r, pl.ds(c, 16)])
                return 0

            lax.fori_loop(0, CH * ng, body, 0)
            pltpu.sync_copy(rA_v, out_hbm.at[pl.ds(t0, CH)])

    return comb(dest, y)


def kernel(x, router_w, router_b, W_up, b_up, W_down, b_down):
    B, N, D = x.shape
    E, _, H = W_up.shape
    T = B * N
    P = 2 * T
    L = P + E * _TG
    n_tiles = L // _TG
    Dw = D // 2

    xb = x.reshape(T, D).astype(jnp.bfloat16)
    xi = lax.bitcast_convert_type(xb.reshape(T, Dw, 2), jnp.int32)
    wub = W_up.astype(jnp.bfloat16)
    wdb = W_down.astype(jnp.bfloat16)

    ep, wp, rk, offs, te, aux = _router(
        xb, router_w.astype(jnp.bfloat16), router_b, E, _TG, n_tiles)

    stok, sw, dest = _dispatch(
        ep.reshape(P), rk.reshape(P), wp.reshape(P), offs.reshape(16), P, T, L)

    xg_i = _gather_rows(stok, xi, L, T, Dw)
    xg = lax.bitcast_convert_type(xg_i, jnp.bfloat16).reshape(L, D)

    y = _ffn(te.reshape(n_tiles), xg, wub, b_up.reshape(E, 1, H),
             wdb, b_down.reshape(E, 1, D), sw.reshape(L, 1),
             L, D, H, E, n_tiles)

    out = _combine(dest, y, T, L, D)

    return out.reshape(B, N, D), aux[0, 0]


# speedup vs baseline: 2.1122x; 2.1122x over previous
import functools

import jax
import jax.numpy as jnp
from jax import lax
from jax.experimental import pallas as pl
from jax.experimental.pallas import tpu as pltpu
from jax.experimental.pallas import tpu_sc as plsc

_TG = 256
_TT = 1024


def _router_body(nt, e_count, t_total, tg, n_tiles,
                 xb_ref, rw_ref, rb_ref,
                 ep_ref, wp_ref, rk_ref, offs_ref, te_ref, aux_ref,
                 carry_ref, fsum_ref, psum_ref):
    s = pl.program_id(0)
    t = pl.program_id(1)
    first = (s == 0) & (t == 0)

    @pl.when(first)
    def _():
        carry_ref[...] = jnp.zeros_like(carry_ref)
        fsum_ref[...] = jnp.zeros_like(fsum_ref)
        psum_ref[...] = jnp.zeros_like(psum_ref)

    xt = xb_ref[...]
    logits = lax.dot_general(
        xt, rw_ref[...], (((1,), (1,)), ((), ())),
        precision=lax.Precision.DEFAULT,
        preferred_element_type=jnp.float32) + rb_ref[...]
    tt, ee = logits.shape
    ids = lax.broadcasted_iota(jnp.int32, (tt, ee), 1)
    m1 = jnp.max(logits, axis=1, keepdims=True)
    a1 = jnp.min(jnp.where(logits == m1, ids, ee), axis=1, keepdims=True)
    sel1 = ids == a1
    rest = jnp.where(sel1, -jnp.inf, logits)
    m2 = jnp.max(rest, axis=1, keepdims=True)
    a2 = jnp.min(jnp.where(rest == m2, ids, ee), axis=1, keepdims=True)
    sel2 = ids == a2
    ez = jnp.where(sel1 | sel2, jnp.exp(logits - m1), 0.0)
    w = ez / jnp.sum(ez, axis=1, keepdims=True)

    @pl.when(s == 0)
    def _():
        fsum_ref[...] += jnp.sum((sel1 | sel2).astype(jnp.float32),
                                 axis=0)[None, :]
        psum_ref[...] += jnp.sum(w, axis=0)[None, :]

    is0 = (s == 0)
    sel = jnp.where(is0, sel1.astype(jnp.float32), sel2.astype(jnp.float32))
    a_slot = jnp.where(is0, a1, a2)
    ep_ref[...] = a_slot
    wp_ref[...] = jnp.sum(sel * w, axis=1, keepdims=True)

    ri = lax.broadcasted_iota(jnp.int32, (tt, tt), 0)
    ci = lax.broadcasted_iota(jnp.int32, (tt, tt), 1)
    tril = (ci < ri).astype(jnp.float32)
    excl = lax.dot_general(
        tril, sel, (((1,), (0,)), ((), ())),
        precision=lax.Precision.HIGHEST,
        preferred_element_type=jnp.float32)
    rank_all = excl + carry_ref[...]
    rank_pair = jnp.sum(rank_all * sel, axis=1, keepdims=True)
    rk_ref[...] = (rank_pair + 0.5).astype(jnp.int32)
    carry_ref[...] += jnp.sum(sel, axis=0)[None, :]

    @pl.when((s == 1) & (t == nt - 1))
    def _fin():
        counts = carry_ref[...]
        pc = jnp.floor((counts + (tg - 1)) * (1.0 / tg)) * tg
        rj = lax.broadcasted_iota(jnp.int32, (ee, 16), 0)
        cj = lax.broadcasted_iota(jnp.int32, (ee, 16), 1)
        upmask = (rj < cj).astype(jnp.float32)
        offs = lax.dot_general(
            pc, upmask, (((1,), (0,)), ((), ())),
            precision=lax.Precision.HIGHEST,
            preferred_element_type=jnp.float32)
        offs_ref[...] = (offs + 0.5).astype(jnp.int32)
        ends = offs[:, :ee] + pc
        tvec = lax.broadcasted_iota(
            jnp.int32, (1, n_tiles), 1).astype(jnp.float32) * tg
        te = jnp.zeros((1, n_tiles), jnp.float32)
        for e in range(e_count):
            te += (ends[:, e:e + 1] <= tvec).astype(jnp.float32)
        te = jnp.minimum(te, e_count - 1)
        te_ref[...] = (te + 0.5).astype(jnp.int32)
        f = fsum_ref[...] * (1.0 / t_total)
        p = psum_ref[...] * (1.0 / t_total)
        aux_ref[...] = e_count * jnp.sum(f * p, keepdims=True)


def _router(xb, router_w, router_b, E, TG, n_tiles):
    T, D = xb.shape
    nt = T // _TT
    grid = (2, nt)
    return pl.pallas_call(
        functools.partial(_router_body, nt, E, float(T), float(TG), n_tiles),
        grid=grid,
        in_specs=[
            pl.BlockSpec((_TT, D), lambda s, t: (t, 0)),
            pl.BlockSpec((E, D), lambda s, t: (0, 0)),
            pl.BlockSpec((1, E), lambda s, t: (0, 0)),
        ],
        out_specs=[
            pl.BlockSpec((_TT, 1), lambda s, t: (s * nt + t, 0)),
            pl.BlockSpec((_TT, 1), lambda s, t: (s * nt + t, 0)),
            pl.BlockSpec((_TT, 1), lambda s, t: (s * nt + t, 0)),
            pl.BlockSpec((1, 16), lambda s, t: (0, 0)),
            pl.BlockSpec((1, n_tiles), lambda s, t: (0, 0)),
            pl.BlockSpec((1, 1), lambda s, t: (0, 0)),
        ],
        out_shape=[
            jax.ShapeDtypeStruct((2 * T, 1), jnp.int32),
            jax.ShapeDtypeStruct((2 * T, 1), jnp.float32),
            jax.ShapeDtypeStruct((2 * T, 1), jnp.int32),
            jax.ShapeDtypeStruct((1, 16), jnp.int32),
            jax.ShapeDtypeStruct((1, n_tiles), jnp.int32),
            jax.ShapeDtypeStruct((1, 1), jnp.float32),
        ],
        scratch_shapes=[pltpu.VMEM((1, E), jnp.float32),
                        pltpu.VMEM((1, E), jnp.float32),
                        pltpu.VMEM((1, E), jnp.float32)],
    )(xb, router_w, router_b.reshape(1, E))


def _dest_body(ep_ref, rk_ref, offs_ref, dest_ref):
    ep = ep_ref[...]
    rk = rk_ref[...]
    tt = ep.shape[0]
    lane = lax.broadcasted_iota(jnp.int32, (tt, 16), 1)
    oh = (lane == ep).astype(jnp.float32)
    offs = offs_ref[...].astype(jnp.float32)
    og = jnp.sum(oh * offs, axis=1, keepdims=True)
    dest_ref[...] = rk + (og + 0.5).astype(jnp.int32)


def _dest_calc(ep, rk, offs, P):
    tile = 1024
    return pl.pallas_call(
        _dest_body,
        grid=(P // tile,),
        in_specs=[
            pl.BlockSpec((tile, 1), lambda t: (t, 0)),
            pl.BlockSpec((tile, 1), lambda t: (t, 0)),
            pl.BlockSpec((1, 16), lambda t: (0, 0)),
        ],
        out_specs=pl.BlockSpec((tile, 1), lambda t: (t, 0)),
        out_shape=jax.ShapeDtypeStruct((P, 1), jnp.int32),
    )(ep, rk, offs)


def _dispatch(dest, wp, P, T, L):
    mesh = plsc.VectorSubcoreMesh(core_axis_name="c", subcore_axis_name="s")
    cpw = P // 32

    @functools.partial(
        pl.kernel, mesh=mesh,
        out_type=[jax.ShapeDtypeStruct((L,), jnp.int32),
                  jax.ShapeDtypeStruct((L,), jnp.float32)],
        scratch_types=[pltpu.VMEM((cpw,), jnp.float32),
                       pltpu.VMEM((2, 128), jnp.int32),
                       pltpu.VMEM((2, 128), jnp.int32),
                       pltpu.SemaphoreType.DMA],
    )
    def disp(dest_hbm, wp_hbm, stok_hbm, sw_hbm,
             wp_v, dest_v, tok_v, sem):
        wid = lax.axis_index("s") * 2 + lax.axis_index("c")
        base = wid * cpw
        tokbase = (wid % 16) * cpw
        pltpu.sync_copy(wp_hbm.at[pl.ds(base, cpw)], wp_v)
        for h in range(2):
            pltpu.sync_copy(dest_hbm.at[pl.ds(base + h * 128, 128)],
                            dest_v.at[h])
            for i in range(8):
                j = h * 8 + i
                tok_v[h, pl.ds(i * 16, 16)] = (
                    lax.iota(jnp.int32, 16) + (tokbase + j * 16))
        cp1 = pltpu.async_copy(tok_v.at[0], stok_hbm.at[dest_v.at[0]], sem)
        cp2 = pltpu.async_copy(tok_v.at[1], stok_hbm.at[dest_v.at[1]], sem)
        cp3 = pltpu.async_copy(wp_v.at[pl.ds(0, 128)],
                               sw_hbm.at[dest_v.at[0]], sem)
        cp4 = pltpu.async_copy(wp_v.at[pl.ds(128, 128)],
                               sw_hbm.at[dest_v.at[1]], sem)
        cp1.wait()
        cp2.wait()
        cp3.wait()
        cp4.wait()

    return disp(dest, wp)


def _gather_rows(stok, x2, L, T, D):
    mesh = plsc.VectorSubcoreMesh(core_axis_name="c", subcore_axis_name="s")
    spw = L // 32
    CH = 16
    NJ = spw // CH

    @functools.partial(
        pl.kernel, mesh=mesh,
        out_type=jax.ShapeDtypeStruct((L, D), jnp.float32),
        scratch_types=[pltpu.VMEM((NJ * CH,), jnp.int32),
                       pltpu.VMEM((CH, D), jnp.float32),
                       pltpu.VMEM((CH, D), jnp.float32),
                       pltpu.SemaphoreType.DMA,
                       pltpu.SemaphoreType.DMA,
                       pltpu.SemaphoreType.DMA,
                       pltpu.SemaphoreType.DMA],
    )
    def gat(stok_hbm, x_hbm, xg_hbm, idx_v, rows0_v, rows1_v,
            gsem0, gsem1, ssem0, ssem1):
        wid = lax.axis_index("s") * 2 + lax.axis_index("c")
        base = wid * spw
        pltpu.sync_copy(stok_hbm.at[pl.ds(base, spw)], idx_v)
        for q in range(spw // 16):
            v = idx_v[pl.ds(q * 16, 16)]
            idx_v[pl.ds(q * 16, 16)] = jnp.minimum(jnp.maximum(v, 0), T - 1)
        rows = (rows0_v, rows1_v)
        gsems = (gsem0, gsem1)
        ssems = (ssem0, ssem1)
        gcp = [None, None]
        scp = [None, None]
        for j in range(NJ):
            b = j % 2
            if scp[b] is not None:
                scp[b].wait()
            gcp[b] = pltpu.async_copy(
                x_hbm.at[idx_v.at[pl.ds(j * CH, CH)]], rows[b], gsems[b])
            if j >= 1:
                pb = (j - 1) % 2
                gcp[pb].wait()
                scp[pb] = pltpu.async_copy(
                    rows[pb], xg_hbm.at[pl.ds(base + (j - 1) * CH, CH)],
                    ssems[pb])
        lb = (NJ - 1) % 2
        gcp[lb].wait()
        pltpu.sync_copy(rows[lb], xg_hbm.at[pl.ds(base + (NJ - 1) * CH, CH)])
        scp[(NJ - 2) % 2].wait()

    return gat(stok, x2)


def _ffn_body(te_ref, xg_ref, wu_ref, bu_ref, wd_ref, bd_ref, w_ref, y_ref):
    h = jnp.dot(xg_ref[...].astype(jnp.bfloat16), wu_ref[0],
                preferred_element_type=jnp.float32)
    h = h + bu_ref[0]
    g = 0.5 * h * (1.0 + lax.erf(h * 0.7071067811865476))
    y = jnp.dot(g.astype(jnp.bfloat16), wd_ref[0],
                preferred_element_type=jnp.float32)
    y_ref[...] = (y + bd_ref[0]) * w_ref[...]


def _ffn(te, xg, wub, bu3, wdb, bd3, sw, L, D, H, E, n_tiles):
    grid_spec = pltpu.PrefetchScalarGridSpec(
        num_scalar_prefetch=1,
        grid=(n_tiles,),
        in_specs=[
            pl.BlockSpec((_TG, D), lambda t, te: (t, 0)),
            pl.BlockSpec((1, D, H), lambda t, te: (te[t], 0, 0)),
            pl.BlockSpec((1, 1, H), lambda t, te: (te[t], 0, 0)),
            pl.BlockSpec((1, H, D), lambda t, te: (te[t], 0, 0)),
            pl.BlockSpec((1, 1, D), lambda t, te: (te[t], 0, 0)),
            pl.BlockSpec((_TG, 1), lambda t, te: (t, 0)),
        ],
        out_specs=pl.BlockSpec((_TG, D), lambda t, te: (t, 0)),
    )
    return pl.pallas_call(
        _ffn_body,
        grid_spec=grid_spec,
        out_shape=jax.ShapeDtypeStruct((L, D), jnp.float32),
    )(te, xg, wub, bu3, wdb, bd3, sw)


def _combine(dest, y, T, L, D):
    mesh = plsc.VectorSubcoreMesh(core_axis_name="c", subcore_axis_name="s")
    tpw = T // 32
    CH = 8

    NJ = tpw // CH
    UN = 32
    ni = (CH * D // 16) // UN

    @functools.partial(
        pl.kernel, mesh=mesh,
        out_type=jax.ShapeDtypeStruct((T, D), jnp.float32),
        scratch_types=[pltpu.VMEM((tpw,), jnp.int32),
                       pltpu.VMEM((tpw,), jnp.int32),
                       pltpu.VMEM((CH, D), jnp.float32),
                       pltpu.VMEM((CH, D), jnp.float32),
                       pltpu.VMEM((CH, D), jnp.float32),
                       pltpu.VMEM((CH, D), jnp.float32),
                       pltpu.SemaphoreType.DMA,
                       pltpu.SemaphoreType.DMA,
                       pltpu.SemaphoreType.DMA,
                       pltpu.SemaphoreType.DMA],
    )
    def comb(dest_hbm, y_hbm, out_hbm, iA_v, iB_v,
             rA0_v, rB0_v, rA1_v, rB1_v, gsem0, gsem1, ssem0, ssem1):
        wid = lax.axis_index("s") * 2 + lax.axis_index("c")
        base = wid * tpw
        pltpu.sync_copy(dest_hbm.at[pl.ds(base, tpw)], iA_v)
        pltpu.sync_copy(dest_hbm.at[pl.ds(T + base, tpw)], iB_v)
        for q in range(tpw // 16):
            sl = pl.ds(q * 16, 16)
            iA_v[sl] = jnp.minimum(jnp.maximum(iA_v[sl], 0), L - 1)
            iB_v[sl] = jnp.minimum(jnp.maximum(iB_v[sl], 0), L - 1)
        rA = (rA0_v, rA1_v)
        rB = (rB0_v, rB1_v)
        gsems = (gsem0, gsem1)
        ssems = (ssem0, ssem1)
        gA = [None, None]
        gB = [None, None]
        scp = [None, None]

        def do_adds(b):
            ra, rb = rA[b], rB[b]
            ngr = D // 16

            def body(i, _):
                for u in range(UN):
                    k = i * UN + u
                    r = k // ngr
                    c = (k % ngr) * 16
                    ra[r, pl.ds(c, 16)] = (ra[r, pl.ds(c, 16)]
                                           + rb[r, pl.ds(c, 16)])
                return 0

            lax.fori_loop(0, ni, body, 0)

        for j in range(NJ):
            b = j % 2
            if scp[b] is not None:
                scp[b].wait()
            gA[b] = pltpu.async_copy(
                y_hbm.at[iA_v.at[pl.ds(j * CH, CH)]], rA[b], gsems[b])
            gB[b] = pltpu.async_copy(
                y_hbm.at[iB_v.at[pl.ds(j * CH, CH)]], rB[b], gsems[b])
            if j >= 1:
                pb = (j - 1) % 2
                gA[pb].wait()
                gB[pb].wait()
                do_adds(pb)
                scp[pb] = pltpu.async_copy(
                    rA[pb], out_hbm.at[pl.ds(base + (j - 1) * CH, CH)],
                    ssems[pb])
        lb = (NJ - 1) % 2
        gA[lb].wait()
        gB[lb].wait()
        do_adds(lb)
        pltpu.sync_copy(rA[lb], out_hbm.at[pl.ds(base + (NJ - 1) * CH, CH)])
        scp[(NJ - 2) % 2].wait()

    return comb(dest, y)


def kernel(x, router_w, router_b, W_up, b_up, W_down, b_down):
    B, N, D = x.shape
    E, _, H = W_up.shape
    T = B * N
    P = 2 * T
    L = P + E * _TG
    n_tiles = L // _TG
    Dw = D // 2

    x2 = x.reshape(T, D)
    wub = W_up.astype(jnp.bfloat16)
    wdb = W_down.astype(jnp.bfloat16)

    ep, wp, rk, offs, te, aux = _router(
        x2, router_w, router_b, E, _TG, n_tiles)

    dest = _dest_calc(ep, rk, offs, P)

    stok, sw = _dispatch(dest.reshape(P), wp.reshape(P), P, T, L)

    xg = _gather_rows(stok, x2, L, T, D)

    y = _ffn(te.reshape(n_tiles), xg, wub, b_up.reshape(E, 1, H),
             wdb, b_down.reshape(E, 1, D), sw.reshape(L, 1),
             L, D, H, E, n_tiles)

    out = _combine(dest.reshape(P), y, T, L, D)

    return out.reshape(B, N, D), aux[0, 0]


# speedup vs baseline: 2.6989x; 1.2778x over previous
import functools

import jax
import jax.numpy as jnp
from jax import lax
from jax.experimental import pallas as pl
from jax.experimental.pallas import tpu as pltpu
from jax.experimental.pallas import tpu_sc as plsc

_TG = 256
_TT = 1024


def _router_body(nt, e_count, t_total, tg, n_tiles,
                 xb_ref, rw_ref, rb_ref,
                 ep_ref, wp_ref, rk_ref, offs_ref, te_ref, aux_ref,
                 carry_ref, fsum_ref, psum_ref):
    s = pl.program_id(0)
    t = pl.program_id(1)
    first = (s == 0) & (t == 0)

    @pl.when(first)
    def _():
        carry_ref[...] = jnp.zeros_like(carry_ref)
        fsum_ref[...] = jnp.zeros_like(fsum_ref)
        psum_ref[...] = jnp.zeros_like(psum_ref)

    xt = xb_ref[...]
    logits = lax.dot_general(
        xt, rw_ref[...], (((1,), (1,)), ((), ())),
        precision=lax.Precision.DEFAULT,
        preferred_element_type=jnp.float32) + rb_ref[...]
    tt, ee = logits.shape
    ids = lax.broadcasted_iota(jnp.int32, (tt, ee), 1)
    m1 = jnp.max(logits, axis=1, keepdims=True)
    a1 = jnp.min(jnp.where(logits == m1, ids, ee), axis=1, keepdims=True)
    sel1 = ids == a1
    rest = jnp.where(sel1, -jnp.inf, logits)
    m2 = jnp.max(rest, axis=1, keepdims=True)
    a2 = jnp.min(jnp.where(rest == m2, ids, ee), axis=1, keepdims=True)
    sel2 = ids == a2
    ez = jnp.where(sel1 | sel2, jnp.exp(logits - m1), 0.0)
    w = ez / jnp.sum(ez, axis=1, keepdims=True)

    @pl.when(s == 0)
    def _():
        fsum_ref[...] += jnp.sum((sel1 | sel2).astype(jnp.float32),
                                 axis=0)[None, :]
        psum_ref[...] += jnp.sum(w, axis=0)[None, :]

    is0 = (s == 0)
    sel = jnp.where(is0, sel1.astype(jnp.float32), sel2.astype(jnp.float32))
    a_slot = jnp.where(is0, a1, a2)
    ep_ref[...] = a_slot
    wp_ref[...] = jnp.sum(sel * w, axis=1, keepdims=True)

    ri = lax.broadcasted_iota(jnp.int32, (tt, tt), 0)
    ci = lax.broadcasted_iota(jnp.int32, (tt, tt), 1)
    tril = (ci < ri).astype(jnp.float32)
    excl = lax.dot_general(
        tril, sel, (((1,), (0,)), ((), ())),
        precision=lax.Precision.HIGHEST,
        preferred_element_type=jnp.float32)
    rank_all = excl + carry_ref[...]
    rank_pair = jnp.sum(rank_all * sel, axis=1, keepdims=True)
    rk_ref[...] = (rank_pair + 0.5).astype(jnp.int32)
    carry_ref[...] += jnp.sum(sel, axis=0)[None, :]

    @pl.when((s == 1) & (t == nt - 1))
    def _fin():
        counts = carry_ref[...]
        pc = jnp.floor((counts + (tg - 1)) * (1.0 / tg)) * tg
        rj = lax.broadcasted_iota(jnp.int32, (ee, 16), 0)
        cj = lax.broadcasted_iota(jnp.int32, (ee, 16), 1)
        upmask = (rj < cj).astype(jnp.float32)
        offs = lax.dot_general(
            pc, upmask, (((1,), (0,)), ((), ())),
            precision=lax.Precision.HIGHEST,
            preferred_element_type=jnp.float32)
        offs_ref[...] = (offs + 0.5).astype(jnp.int32)
        ends = offs[:, :ee] + pc
        tvec = lax.broadcasted_iota(
            jnp.int32, (1, n_tiles), 1).astype(jnp.float32) * tg
        te = jnp.zeros((1, n_tiles), jnp.float32)
        for e in range(e_count):
            te += (ends[:, e:e + 1] <= tvec).astype(jnp.float32)
        te = jnp.minimum(te, e_count - 1)
        te_ref[...] = (te + 0.5).astype(jnp.int32)
        f = fsum_ref[...] * (1.0 / t_total)
        p = psum_ref[...] * (1.0 / t_total)
        aux_ref[...] = e_count * jnp.sum(f * p, keepdims=True)


def _router(xb, router_w, router_b, E, TG, n_tiles):
    T, D = xb.shape
    nt = T // _TT
    grid = (2, nt)
    return pl.pallas_call(
        functools.partial(_router_body, nt, E, float(T), float(TG), n_tiles),
        grid=grid,
        in_specs=[
            pl.BlockSpec((_TT, D), lambda s, t: (t, 0)),
            pl.BlockSpec((E, D), lambda s, t: (0, 0)),
            pl.BlockSpec((1, E), lambda s, t: (0, 0)),
        ],
        out_specs=[
            pl.BlockSpec((_TT, 1), lambda s, t: (s * nt + t, 0)),
            pl.BlockSpec((_TT, 1), lambda s, t: (s * nt + t, 0)),
            pl.BlockSpec((_TT, 1), lambda s, t: (s * nt + t, 0)),
            pl.BlockSpec((1, 16), lambda s, t: (0, 0)),
            pl.BlockSpec((1, n_tiles), lambda s, t: (0, 0)),
            pl.BlockSpec((1, 1), lambda s, t: (0, 0)),
        ],
        out_shape=[
            jax.ShapeDtypeStruct((2 * T, 1), jnp.int32),
            jax.ShapeDtypeStruct((2 * T, 1), jnp.float32),
            jax.ShapeDtypeStruct((2 * T, 1), jnp.int32),
            jax.ShapeDtypeStruct((1, 16), jnp.int32),
            jax.ShapeDtypeStruct((1, n_tiles), jnp.int32),
            jax.ShapeDtypeStruct((1, 1), jnp.float32),
        ],
        scratch_shapes=[pltpu.VMEM((1, E), jnp.float32),
                        pltpu.VMEM((1, E), jnp.float32),
                        pltpu.VMEM((1, E), jnp.float32)],
    )(xb, router_w, router_b.reshape(1, E))


def _dest_body(ep_ref, rk_ref, offs_ref, dest_ref):
    ep = ep_ref[...]
    rk = rk_ref[...]
    tt = ep.shape[0]
    lane = lax.broadcasted_iota(jnp.int32, (tt, 16), 1)
    oh = (lane == ep).astype(jnp.float32)
    offs = offs_ref[...].astype(jnp.float32)
    og = jnp.sum(oh * offs, axis=1, keepdims=True)
    dest_ref[...] = rk + (og + 0.5).astype(jnp.int32)


def _dest_calc(ep, rk, offs, P):
    tile = 1024
    return pl.pallas_call(
        _dest_body,
        grid=(P // tile,),
        in_specs=[
            pl.BlockSpec((tile, 1), lambda t: (t, 0)),
            pl.BlockSpec((tile, 1), lambda t: (t, 0)),
            pl.BlockSpec((1, 16), lambda t: (0, 0)),
        ],
        out_specs=pl.BlockSpec((tile, 1), lambda t: (t, 0)),
        out_shape=jax.ShapeDtypeStruct((P, 1), jnp.int32),
    )(ep, rk, offs)


def _scatter_rows(dest, wp, x2, P, T, L, D):
    mesh = plsc.VectorSubcoreMesh(core_axis_name="c", subcore_axis_name="s")
    cpw = P // 32
    CH = 16
    NJ = cpw // CH

    @functools.partial(
        pl.kernel, mesh=mesh,
        out_type=[jax.ShapeDtypeStruct((L, D), jnp.float32),
                  jax.ShapeDtypeStruct((L,), jnp.float32)],
        scratch_types=[pltpu.VMEM((NJ, CH), jnp.int32),
                       pltpu.VMEM((2, 128), jnp.int32),
                       pltpu.VMEM((cpw,), jnp.float32),
                       pltpu.VMEM((CH, D), jnp.float32),
                       pltpu.VMEM((CH, D), jnp.float32),
                       pltpu.VMEM((CH, D), jnp.float32),
                       pltpu.SemaphoreType.DMA,
                       pltpu.SemaphoreType.DMA,
                       pltpu.SemaphoreType.DMA,
                       pltpu.SemaphoreType.DMA,
                       pltpu.SemaphoreType.DMA,
                       pltpu.SemaphoreType.DMA,
                       pltpu.SemaphoreType.DMA],
    )
    def scat(dest_hbm, wp_hbm, x_hbm, xg_hbm, sw_hbm,
             didx_v, dw_v, wp_v, r0_v, r1_v, r2_v,
             rs0, rs1, rs2, ss0, ss1, ss2, wsem):
        wid = lax.axis_index("s") * 2 + lax.axis_index("c")
        base = wid * cpw
        tokbase = (wid % 16) * cpw
        for j in range(NJ):
            pltpu.sync_copy(dest_hbm.at[pl.ds(base + j * CH, CH)],
                            didx_v.at[j])
        for q in range(NJ):
            v = didx_v[q, pl.ds(0, 16)]
            didx_v[q, pl.ds(0, 16)] = jnp.minimum(jnp.maximum(v, 0), L - 1)
        pltpu.sync_copy(wp_hbm.at[pl.ds(base, cpw)], wp_v)
        for h in range(2):
            pltpu.sync_copy(dest_hbm.at[pl.ds(base + h * 128, 128)],
                            dw_v.at[h])
        w1 = pltpu.async_copy(wp_v.at[pl.ds(0, 128)],
                              sw_hbm.at[dw_v.at[0]], wsem)
        w2 = pltpu.async_copy(wp_v.at[pl.ds(128, 128)],
                              sw_hbm.at[dw_v.at[1]], wsem)
        rows = (r0_v, r1_v, r2_v)
        rsems = (rs0, rs1, rs2)
        ssems = (ss0, ss1, ss2)
        rcp = [None, None, None]
        scp = [None, None, None]
        for j in range(NJ):
            b = j % 3
            if scp[b] is not None:
                scp[b].wait()
            rcp[b] = pltpu.async_copy(
                x_hbm.at[pl.ds(tokbase + j * CH, CH)], rows[b], rsems[b])
            if j >= 1:
                pb = (j - 1) % 3
                rcp[pb].wait()
                scp[pb] = pltpu.async_copy(
                    rows[pb], xg_hbm.at[didx_v.at[j - 1]], ssems[pb])
        lb = (NJ - 1) % 3
        rcp[lb].wait()
        scp[lb] = pltpu.async_copy(
            rows[lb], xg_hbm.at[didx_v.at[NJ - 1]], ssems[lb])
        scp[lb].wait()
        scp[(NJ - 2) % 3].wait()
        scp[(NJ - 3) % 3].wait()
        w1.wait()
        w2.wait()

    return scat(dest, wp, x2)


def _ffn_body(te_ref, xg_ref, wu_ref, bu_ref, wd_ref, bd_ref, w_ref, y_ref):
    h = jnp.dot(xg_ref[...].astype(jnp.bfloat16), wu_ref[0],
                preferred_element_type=jnp.float32)
    h = h + bu_ref[0]
    g = 0.5 * h * (1.0 + lax.erf(h * 0.7071067811865476))
    y = jnp.dot(g.astype(jnp.bfloat16), wd_ref[0],
                preferred_element_type=jnp.float32)
    y_ref[...] = (y + bd_ref[0]) * w_ref[...]


def _ffn(te, xg, wub, bu3, wdb, bd3, sw, L, D, H, E, n_tiles):
    grid_spec = pltpu.PrefetchScalarGridSpec(
        num_scalar_prefetch=1,
        grid=(n_tiles,),
        in_specs=[
            pl.BlockSpec((_TG, D), lambda t, te: (t, 0)),
            pl.BlockSpec((1, D, H), lambda t, te: (te[t], 0, 0)),
            pl.BlockSpec((1, 1, H), lambda t, te: (te[t], 0, 0)),
            pl.BlockSpec((1, H, D), lambda t, te: (te[t], 0, 0)),
            pl.BlockSpec((1, 1, D), lambda t, te: (te[t], 0, 0)),
            pl.BlockSpec((_TG, 1), lambda t, te: (t, 0)),
        ],
        out_specs=pl.BlockSpec((_TG, D), lambda t, te: (t, 0)),
    )
    return pl.pallas_call(
        _ffn_body,
        grid_spec=grid_spec,
        out_shape=jax.ShapeDtypeStruct((L, D), jnp.float32),
    )(te, xg, wub, bu3, wdb, bd3, sw)


def _combine(dest, y, T, L, D):
    mesh = plsc.VectorSubcoreMesh(core_axis_name="c", subcore_axis_name="s")
    tpw = T // 32
    CH = 8

    NJ = tpw // CH
    UN = 32
    ni = (CH * D // 16) // UN

    @functools.partial(
        pl.kernel, mesh=mesh,
        out_type=jax.ShapeDtypeStruct((T, D), jnp.float32),
        scratch_types=[pltpu.VMEM((tpw,), jnp.int32),
                       pltpu.VMEM((tpw,), jnp.int32),
                       pltpu.VMEM((CH, D), jnp.float32),
                       pltpu.VMEM((CH, D), jnp.float32),
                       pltpu.VMEM((CH, D), jnp.float32),
                       pltpu.VMEM((CH, D), jnp.float32),
                       pltpu.SemaphoreType.DMA,
                       pltpu.SemaphoreType.DMA,
                       pltpu.SemaphoreType.DMA,
                       pltpu.SemaphoreType.DMA],
    )
    def comb(dest_hbm, y_hbm, out_hbm, iA_v, iB_v,
             rA0_v, rB0_v, rA1_v, rB1_v, gsem0, gsem1, ssem0, ssem1):
        wid = lax.axis_index("s") * 2 + lax.axis_index("c")
        base = wid * tpw
        pltpu.sync_copy(dest_hbm.at[pl.ds(base, tpw)], iA_v)
        pltpu.sync_copy(dest_hbm.at[pl.ds(T + base, tpw)], iB_v)
        for q in range(tpw // 16):
            sl = pl.ds(q * 16, 16)
            iA_v[sl] = jnp.minimum(jnp.maximum(iA_v[sl], 0), L - 1)
            iB_v[sl] = jnp.minimum(jnp.maximum(iB_v[sl], 0), L - 1)
        rA = (rA0_v, rA1_v)
        rB = (rB0_v, rB1_v)
        gsems = (gsem0, gsem1)
        ssems = (ssem0, ssem1)
        gA = [None, None]
        gB = [None, None]
        scp = [None, None]

        def do_adds(b):
            ra, rb = rA[b], rB[b]
            ngr = D // 16

            def body(i, _):
                for u in range(UN):
                    k = i * UN + u
                    r = k // ngr
                    c = (k % ngr) * 16
                    ra[r, pl.ds(c, 16)] = (ra[r, pl.ds(c, 16)]
                                           + rb[r, pl.ds(c, 16)])
                return 0

            lax.fori_loop(0, ni, body, 0)

        for j in range(NJ):
            b = j % 2
            if scp[b] is not None:
                scp[b].wait()
            gA[b] = pltpu.async_copy(
                y_hbm.at[iA_v.at[pl.ds(j * CH, CH)]], rA[b], gsems[b])
            gB[b] = pltpu.async_copy(
                y_hbm.at[iB_v.at[pl.ds(j * CH, CH)]], rB[b], gsems[b])
            if j >= 1:
                pb = (j - 1) % 2
                gA[pb].wait()
                gB[pb].wait()
                do_adds(pb)
                scp[pb] = pltpu.async_copy(
                    rA[pb], out_hbm.at[pl.ds(base + (j - 1) * CH, CH)],
                    ssems[pb])
        lb = (NJ - 1) % 2
        gA[lb].wait()
        gB[lb].wait()
        do_adds(lb)
        pltpu.sync_copy(rA[lb], out_hbm.at[pl.ds(base + (NJ - 1) * CH, CH)])
        scp[(NJ - 2) % 2].wait()

    return comb(dest, y)


def kernel(x, router_w, router_b, W_up, b_up, W_down, b_down):
    B, N, D = x.shape
    E, _, H = W_up.shape
    T = B * N
    P = 2 * T
    L = P + E * _TG
    n_tiles = L // _TG
    Dw = D // 2

    x2 = x.reshape(T, D)
    wub = W_up.astype(jnp.bfloat16)
    wdb = W_down.astype(jnp.bfloat16)

    ep, wp, rk, offs, te, aux = _router(
        x2, router_w, router_b, E, _TG, n_tiles)

    dest = _dest_calc(ep, rk, offs, P)

    xg, sw = _scatter_rows(dest.reshape(P), wp.reshape(P), x2, P, T, L, D)

    y = _ffn(te.reshape(n_tiles), xg, wub, b_up.reshape(E, 1, H),
             wdb, b_down.reshape(E, 1, D), sw.reshape(L, 1),
             L, D, H, E, n_tiles)

    out = _combine(dest.reshape(P), y, T, L, D)

    return out.reshape(B, N, D), aux[0, 0]


# speedup vs baseline: 2.8778x; 1.0663x over previous
import functools

import jax
import jax.numpy as jnp
from jax import lax
from jax.experimental import pallas as pl
from jax.experimental.pallas import tpu as pltpu
from jax.experimental.pallas import tpu_sc as plsc

_TG = 256
_TT = 1024


def _router_body(nt, e_count, t_total, tg, n_tiles,
                 xb_ref, rw_ref, rb_ref,
                 ep_ref, wp_ref, rk_ref, offs_ref, te_ref, aux_ref,
                 carry_ref, fsum_ref, psum_ref, tril_ref):
    s = pl.program_id(0)
    t = pl.program_id(1)
    first = (s == 0) & (t == 0)

    @pl.when(first)
    def _():
        carry_ref[...] = jnp.zeros_like(carry_ref)
        fsum_ref[...] = jnp.zeros_like(fsum_ref)
        psum_ref[...] = jnp.zeros_like(psum_ref)

    xt = xb_ref[...]
    logits = lax.dot_general(
        xt, rw_ref[...], (((1,), (1,)), ((), ())),
        precision=lax.Precision.DEFAULT,
        preferred_element_type=jnp.float32) + rb_ref[...]
    tt, ee = logits.shape
    ids = lax.broadcasted_iota(jnp.int32, (tt, ee), 1)
    m1 = jnp.max(logits, axis=1, keepdims=True)
    a1 = jnp.min(jnp.where(logits == m1, ids, ee), axis=1, keepdims=True)
    sel1 = ids == a1
    rest = jnp.where(sel1, -jnp.inf, logits)
    m2 = jnp.max(rest, axis=1, keepdims=True)
    a2 = jnp.min(jnp.where(rest == m2, ids, ee), axis=1, keepdims=True)
    sel2 = ids == a2
    ez = jnp.where(sel1 | sel2, jnp.exp(logits - m1), 0.0)
    w = ez / jnp.sum(ez, axis=1, keepdims=True)

    @pl.when(s == 0)
    def _():
        fsum_ref[...] += jnp.sum((sel1 | sel2).astype(jnp.float32),
                                 axis=0)[None, :]
        psum_ref[...] += jnp.sum(w, axis=0)[None, :]

    is0 = (s == 0)
    sel = jnp.where(is0, sel1.astype(jnp.float32), sel2.astype(jnp.float32))
    a_slot = jnp.where(is0, a1, a2)
    ep_ref[...] = a_slot
    wp_ref[...] = jnp.sum(sel * w, axis=1, keepdims=True)

    @pl.when(first)
    def _():
        ri = lax.broadcasted_iota(jnp.int32, (tt, tt), 0)
        ci = lax.broadcasted_iota(jnp.int32, (tt, tt), 1)
        tril_ref[...] = (ci < ri).astype(jnp.float32)

    excl = lax.dot_general(
        tril_ref[...], sel, (((1,), (0,)), ((), ())),
        precision=lax.Precision.DEFAULT,
        preferred_element_type=jnp.float32)
    rank_all = excl + carry_ref[...]
    rank_pair = jnp.sum(rank_all * sel, axis=1, keepdims=True)
    rk_ref[...] = (rank_pair + 0.5).astype(jnp.int32)
    carry_ref[...] += jnp.sum(sel, axis=0)[None, :]

    @pl.when((s == 1) & (t == nt - 1))
    def _fin():
        counts = carry_ref[...]
        pc = jnp.floor((counts + (tg - 1)) * (1.0 / tg)) * tg
        rj = lax.broadcasted_iota(jnp.int32, (ee, 16), 0)
        cj = lax.broadcasted_iota(jnp.int32, (ee, 16), 1)
        upmask = (rj < cj).astype(jnp.float32)
        offs = lax.dot_general(
            pc, upmask, (((1,), (0,)), ((), ())),
            precision=lax.Precision.HIGHEST,
            preferred_element_type=jnp.float32)
        offs_ref[...] = (offs + 0.5).astype(jnp.int32)
        ends = offs[:, :ee] + pc
        tvec = lax.broadcasted_iota(
            jnp.int32, (1, n_tiles), 1).astype(jnp.float32) * tg
        te = jnp.zeros((1, n_tiles), jnp.float32)
        for e in range(e_count):
            te += (ends[:, e:e + 1] <= tvec).astype(jnp.float32)
        te = jnp.minimum(te, e_count - 1)
        te_ref[...] = (te + 0.5).astype(jnp.int32)
        f = fsum_ref[...] * (1.0 / t_total)
        p = psum_ref[...] * (1.0 / t_total)
        aux_ref[...] = e_count * jnp.sum(f * p, keepdims=True)


def _router(xb, router_w, router_b, E, TG, n_tiles):
    T, D = xb.shape
    nt = T // _TT
    grid = (2, nt)
    return pl.pallas_call(
        functools.partial(_router_body, nt, E, float(T), float(TG), n_tiles),
        grid=grid,
        in_specs=[
            pl.BlockSpec((_TT, D), lambda s, t: (t, 0)),
            pl.BlockSpec((E, D), lambda s, t: (0, 0)),
            pl.BlockSpec((1, E), lambda s, t: (0, 0)),
        ],
        out_specs=[
            pl.BlockSpec((_TT, 1), lambda s, t: (s * nt + t, 0)),
            pl.BlockSpec((_TT, 1), lambda s, t: (s * nt + t, 0)),
            pl.BlockSpec((_TT, 1), lambda s, t: (s * nt + t, 0)),
            pl.BlockSpec((1, 16), lambda s, t: (0, 0)),
            pl.BlockSpec((1, n_tiles), lambda s, t: (0, 0)),
            pl.BlockSpec((1, 1), lambda s, t: (0, 0)),
        ],
        out_shape=[
            jax.ShapeDtypeStruct((2 * T, 1), jnp.int32),
            jax.ShapeDtypeStruct((2 * T, 1), jnp.float32),
            jax.ShapeDtypeStruct((2 * T, 1), jnp.int32),
            jax.ShapeDtypeStruct((1, 16), jnp.int32),
            jax.ShapeDtypeStruct((1, n_tiles), jnp.int32),
            jax.ShapeDtypeStruct((1, 1), jnp.float32),
        ],
        scratch_shapes=[pltpu.VMEM((1, E), jnp.float32),
                        pltpu.VMEM((1, E), jnp.float32),
                        pltpu.VMEM((1, E), jnp.float32),
                        pltpu.VMEM((_TT, _TT), jnp.float32)],
    )(xb, router_w, router_b.reshape(1, E))


def _dest_body(ep_ref, rk_ref, offs_ref, dest_ref):
    ep = ep_ref[...]
    rk = rk_ref[...]
    tt = ep.shape[0]
    lane = lax.broadcasted_iota(jnp.int32, (tt, 16), 1)
    oh = (lane == ep).astype(jnp.float32)
    offs = offs_ref[...].astype(jnp.float32)
    og = jnp.sum(oh * offs, axis=1, keepdims=True)
    dest_ref[...] = rk + (og + 0.5).astype(jnp.int32)


def _dest_calc(ep, rk, offs, P):
    tile = 1024
    return pl.pallas_call(
        _dest_body,
        grid=(P // tile,),
        in_specs=[
            pl.BlockSpec((tile, 1), lambda t: (t, 0)),
            pl.BlockSpec((tile, 1), lambda t: (t, 0)),
            pl.BlockSpec((1, 16), lambda t: (0, 0)),
        ],
        out_specs=pl.BlockSpec((tile, 1), lambda t: (t, 0)),
        out_shape=jax.ShapeDtypeStruct((P, 1), jnp.int32),
    )(ep, rk, offs)


def _scatter_rows(dest, wp, x2, P, T, L, D):
    mesh = plsc.VectorSubcoreMesh(core_axis_name="c", subcore_axis_name="s")
    cpw = P // 32
    CH = 16
    NJ = cpw // CH

    @functools.partial(
        pl.kernel, mesh=mesh,
        out_type=[jax.ShapeDtypeStruct((L, D), jnp.float32),
                  jax.ShapeDtypeStruct((L,), jnp.float32)],
        scratch_types=[pltpu.VMEM((NJ, CH), jnp.int32),
                       pltpu.VMEM((2, 128), jnp.int32),
                       pltpu.VMEM((cpw,), jnp.float32),
                       pltpu.VMEM((CH, D), jnp.float32),
                       pltpu.VMEM((CH, D), jnp.float32),
                       pltpu.VMEM((CH, D), jnp.float32),
                       pltpu.SemaphoreType.DMA,
                       pltpu.SemaphoreType.DMA,
                       pltpu.SemaphoreType.DMA,
                       pltpu.SemaphoreType.DMA,
                       pltpu.SemaphoreType.DMA,
                       pltpu.SemaphoreType.DMA,
                       pltpu.SemaphoreType.DMA],
    )
    def scat(dest_hbm, wp_hbm, x_hbm, xg_hbm, sw_hbm,
             didx_v, dw_v, wp_v, r0_v, r1_v, r2_v,
             rs0, rs1, rs2, ss0, ss1, ss2, wsem):
        wid = lax.axis_index("s") * 2 + lax.axis_index("c")
        base = wid * cpw
        tokbase = (wid % 16) * cpw
        for j in range(NJ):
            pltpu.sync_copy(dest_hbm.at[pl.ds(base + j * CH, CH)],
                            didx_v.at[j])
        for q in range(NJ):
            v = didx_v[q, pl.ds(0, 16)]
            didx_v[q, pl.ds(0, 16)] = jnp.minimum(jnp.maximum(v, 0), L - 1)
        pltpu.sync_copy(wp_hbm.at[pl.ds(base, cpw)], wp_v)
        for h in range(2):
            pltpu.sync_copy(dest_hbm.at[pl.ds(base + h * 128, 128)],
                            dw_v.at[h])
        w1 = pltpu.async_copy(wp_v.at[pl.ds(0, 128)],
                              sw_hbm.at[dw_v.at[0]], wsem)
        w2 = pltpu.async_copy(wp_v.at[pl.ds(128, 128)],
                              sw_hbm.at[dw_v.at[1]], wsem)
        rows = (r0_v, r1_v, r2_v)
        rsems = (rs0, rs1, rs2)
        ssems = (ss0, ss1, ss2)
        rcp = [None, None, None]
        scp = [None, None, None]
        for j in range(NJ):
            b = j % 3
            if scp[b] is not None:
                scp[b].wait()
            rcp[b] = pltpu.async_copy(
                x_hbm.at[pl.ds(tokbase + j * CH, CH)], rows[b], rsems[b])
            if j >= 1:
                pb = (j - 1) % 3
                rcp[pb].wait()
                scp[pb] = pltpu.async_copy(
                    rows[pb], xg_hbm.at[didx_v.at[j - 1]], ssems[pb])
        lb = (NJ - 1) % 3
        rcp[lb].wait()
        scp[lb] = pltpu.async_copy(
            rows[lb], xg_hbm.at[didx_v.at[NJ - 1]], ssems[lb])
        scp[lb].wait()
        scp[(NJ - 2) % 3].wait()
        scp[(NJ - 3) % 3].wait()
        w1.wait()
        w2.wait()

    return scat(dest, wp, x2)


def _ffn_body(te_ref, xg_ref, wu_ref, bu_ref, wd_ref, bd_ref, w_ref, y_ref):
    h = jnp.dot(xg_ref[...].astype(jnp.bfloat16), wu_ref[0],
                preferred_element_type=jnp.float32)
    h = h + bu_ref[0]
    g = 0.5 * h * (1.0 + lax.erf(h * 0.7071067811865476))
    y = jnp.dot(g.astype(jnp.bfloat16), wd_ref[0],
                preferred_element_type=jnp.float32)
    y_ref[...] = (y + bd_ref[0]) * w_ref[...]


def _ffn(te, xg, wub, bu3, wdb, bd3, sw, L, D, H, E, n_tiles):
    grid_spec = pltpu.PrefetchScalarGridSpec(
        num_scalar_prefetch=1,
        grid=(n_tiles,),
        in_specs=[
            pl.BlockSpec((_TG, D), lambda t, te: (t, 0)),
            pl.BlockSpec((1, D, H), lambda t, te: (te[t], 0, 0)),
            pl.BlockSpec((1, 1, H), lambda t, te: (te[t], 0, 0)),
            pl.BlockSpec((1, H, D), lambda t, te: (te[t], 0, 0)),
            pl.BlockSpec((1, 1, D), lambda t, te: (te[t], 0, 0)),
            pl.BlockSpec((_TG, 1), lambda t, te: (t, 0)),
        ],
        out_specs=pl.BlockSpec((_TG, D), lambda t, te: (t, 0)),
    )
    return pl.pallas_call(
        _ffn_body,
        grid_spec=grid_spec,
        out_shape=jax.ShapeDtypeStruct((L, D), jnp.float32),
    )(te, xg, wub, bu3, wdb, bd3, sw)


def _combine(dest, y, T, L, D):
    mesh = plsc.VectorSubcoreMesh(core_axis_name="c", subcore_axis_name="s")
    tpw = T // 32
    CH = 8

    NJ = tpw // CH
    UN = 32
    ni = (CH * D // 16) // UN

    @functools.partial(
        pl.kernel, mesh=mesh,
        out_type=jax.ShapeDtypeStruct((T, D), jnp.float32),
        scratch_types=[pltpu.VMEM((tpw,), jnp.int32),
                       pltpu.VMEM((tpw,), jnp.int32),
                       pltpu.VMEM((CH, D), jnp.float32),
                       pltpu.VMEM((CH, D), jnp.float32),
                       pltpu.VMEM((CH, D), jnp.float32),
                       pltpu.VMEM((CH, D), jnp.float32),
                       pltpu.SemaphoreType.DMA,
                       pltpu.SemaphoreType.DMA,
                       pltpu.SemaphoreType.DMA,
                       pltpu.SemaphoreType.DMA],
    )
    def comb(dest_hbm, y_hbm, out_hbm, iA_v, iB_v,
             rA0_v, rB0_v, rA1_v, rB1_v, gsem0, gsem1, ssem0, ssem1):
        wid = lax.axis_index("s") * 2 + lax.axis_index("c")
        base = wid * tpw
        pltpu.sync_copy(dest_hbm.at[pl.ds(base, tpw)], iA_v)
        pltpu.sync_copy(dest_hbm.at[pl.ds(T + base, tpw)], iB_v)
        for q in range(tpw // 16):
            sl = pl.ds(q * 16, 16)
            iA_v[sl] = jnp.minimum(jnp.maximum(iA_v[sl], 0), L - 1)
            iB_v[sl] = jnp.minimum(jnp.maximum(iB_v[sl], 0), L - 1)
        rA = (rA0_v, rA1_v)
        rB = (rB0_v, rB1_v)
        gsems = (gsem0, gsem1)
        ssems = (ssem0, ssem1)
        gA = [None, None]
        gB = [None, None]
        scp = [None, None]

        def do_adds(b):
            ra, rb = rA[b], rB[b]
            ngr = D // 16

            def body(i, _):
                for u in range(UN):
                    k = i * UN + u
                    r = k // ngr
                    c = (k % ngr) * 16
                    ra[r, pl.ds(c, 16)] = (ra[r, pl.ds(c, 16)]
                                           + rb[r, pl.ds(c, 16)])
                return 0

            lax.fori_loop(0, ni, body, 0)

        for j in range(NJ):
            b = j % 2
            if scp[b] is not None:
                scp[b].wait()
            gA[b] = pltpu.async_copy(
                y_hbm.at[iA_v.at[pl.ds(j * CH, CH)]], rA[b], gsems[b])
            gB[b] = pltpu.async_copy(
                y_hbm.at[iB_v.at[pl.ds(j * CH, CH)]], rB[b], gsems[b])
            if j >= 1:
                pb = (j - 1) % 2
                gA[pb].wait()
                gB[pb].wait()
                do_adds(pb)
                scp[pb] = pltpu.async_copy(
                    rA[pb], out_hbm.at[pl.ds(base + (j - 1) * CH, CH)],
                    ssems[pb])
        lb = (NJ - 1) % 2
        gA[lb].wait()
        gB[lb].wait()
        do_adds(lb)
        pltpu.sync_copy(rA[lb], out_hbm.at[pl.ds(base + (NJ - 1) * CH, CH)])
        scp[(NJ - 2) % 2].wait()

    return comb(dest, y)


def kernel(x, router_w, router_b, W_up, b_up, W_down, b_down):
    B, N, D = x.shape
    E, _, H = W_up.shape
    T = B * N
    P = 2 * T
    L = P + E * _TG
    n_tiles = L // _TG
    Dw = D // 2

    x2 = x.reshape(T, D)
    wub = W_up.astype(jnp.bfloat16)
    wdb = W_down.astype(jnp.bfloat16)

    ep, wp, rk, offs, te, aux = _router(
        x2, router_w, router_b, E, _TG, n_tiles)

    dest = _dest_calc(ep, rk, offs, P)

    xg, sw = _scatter_rows(dest.reshape(P), wp.reshape(P), x2, P, T, L, D)

    y = _ffn(te.reshape(n_tiles), xg, wub, b_up.reshape(E, 1, H),
             wdb, b_down.reshape(E, 1, D), sw.reshape(L, 1),
             L, D, H, E, n_tiles)

    out = _combine(dest.reshape(P), y, T, L, D)

    return out.reshape(B, N, D), aux[0, 0]


# speedup vs baseline: 2.9376x; 1.0208x over previous
import functools

import jax
import jax.numpy as jnp
from jax import lax
from jax.experimental import pallas as pl
from jax.experimental.pallas import tpu as pltpu
from jax.experimental.pallas import tpu_sc as plsc

_TG = 256
_TT = 1024


def _router_body(nt, e_count, t_total, tg, n_tiles,
                 xb_ref, rw_ref, rb_ref,
                 ep_ref, wp_ref, rk_ref, offs_ref, te_ref, aux_ref,
                 carry_ref, fsum_ref, psum_ref, tril_ref):
    s = pl.program_id(0)
    t = pl.program_id(1)
    first = (s == 0) & (t == 0)

    @pl.when(first)
    def _():
        carry_ref[...] = jnp.zeros_like(carry_ref)
        fsum_ref[...] = jnp.zeros_like(fsum_ref)
        psum_ref[...] = jnp.zeros_like(psum_ref)

    xt = xb_ref[...]
    logits = lax.dot_general(
        xt, rw_ref[...], (((1,), (1,)), ((), ())),
        precision=lax.Precision.DEFAULT,
        preferred_element_type=jnp.float32) + rb_ref[...]
    tt, ee = logits.shape
    ids = lax.broadcasted_iota(jnp.int32, (tt, ee), 1)
    m1 = jnp.max(logits, axis=1, keepdims=True)
    a1 = jnp.min(jnp.where(logits == m1, ids, ee), axis=1, keepdims=True)
    sel1 = ids == a1
    rest = jnp.where(sel1, -jnp.inf, logits)
    m2 = jnp.max(rest, axis=1, keepdims=True)
    a2 = jnp.min(jnp.where(rest == m2, ids, ee), axis=1, keepdims=True)
    sel2 = ids == a2
    ez = jnp.where(sel1 | sel2, jnp.exp(logits - m1), 0.0)
    w = ez / jnp.sum(ez, axis=1, keepdims=True)

    @pl.when(s == 0)
    def _():
        fsum_ref[...] += jnp.sum((sel1 | sel2).astype(jnp.float32),
                                 axis=0)[None, :]
        psum_ref[...] += jnp.sum(w, axis=0)[None, :]

    is0 = (s == 0)
    sel = jnp.where(is0, sel1.astype(jnp.float32), sel2.astype(jnp.float32))
    a_slot = jnp.where(is0, a1, a2)
    ep_ref[...] = a_slot
    wp_ref[...] = jnp.sum(sel * w, axis=1, keepdims=True)

    @pl.when(first)
    def _():
        ri = lax.broadcasted_iota(jnp.int32, (tt, tt), 0)
        ci = lax.broadcasted_iota(jnp.int32, (tt, tt), 1)
        tril_ref[...] = (ci < ri).astype(jnp.float32)

    excl = lax.dot_general(
        tril_ref[...], sel, (((1,), (0,)), ((), ())),
        precision=lax.Precision.DEFAULT,
        preferred_element_type=jnp.float32)
    rank_all = excl + carry_ref[...]
    rank_pair = jnp.sum(rank_all * sel, axis=1, keepdims=True)
    rk_ref[...] = (rank_pair + 0.5).astype(jnp.int32)
    carry_ref[...] += jnp.sum(sel, axis=0)[None, :]

    @pl.when((s == 1) & (t == nt - 1))
    def _fin():
        counts = carry_ref[...]
        pc = jnp.floor((counts + (tg - 1)) * (1.0 / tg)) * tg
        rj = lax.broadcasted_iota(jnp.int32, (ee, 16), 0)
        cj = lax.broadcasted_iota(jnp.int32, (ee, 16), 1)
        upmask = (rj < cj).astype(jnp.float32)
        offs = lax.dot_general(
            pc, upmask, (((1,), (0,)), ((), ())),
            precision=lax.Precision.HIGHEST,
            preferred_element_type=jnp.float32)
        offs_ref[...] = (offs + 0.5).astype(jnp.int32)
        ends = offs[:, :ee] + pc
        tvec = lax.broadcasted_iota(
            jnp.int32, (1, n_tiles), 1).astype(jnp.float32) * tg
        te = jnp.zeros((1, n_tiles), jnp.float32)
        for e in range(e_count):
            te += (ends[:, e:e + 1] <= tvec).astype(jnp.float32)
        te = jnp.minimum(te, e_count - 1)
        te_ref[...] = (te + 0.5).astype(jnp.int32)
        f = fsum_ref[...] * (1.0 / t_total)
        p = psum_ref[...] * (1.0 / t_total)
        aux_ref[...] = e_count * jnp.sum(f * p, keepdims=True)


def _router(xb, router_w, router_b, E, TG, n_tiles):
    T, D = xb.shape
    nt = T // _TT
    grid = (2, nt)
    return pl.pallas_call(
        functools.partial(_router_body, nt, E, float(T), float(TG), n_tiles),
        grid=grid,
        in_specs=[
            pl.BlockSpec((_TT, D), lambda s, t: (t, 0)),
            pl.BlockSpec((E, D), lambda s, t: (0, 0)),
            pl.BlockSpec((1, E), lambda s, t: (0, 0)),
        ],
        out_specs=[
            pl.BlockSpec((_TT, 1), lambda s, t: (s * nt + t, 0)),
            pl.BlockSpec((_TT, 1), lambda s, t: (s * nt + t, 0)),
            pl.BlockSpec((_TT, 1), lambda s, t: (s * nt + t, 0)),
            pl.BlockSpec((1, 16), lambda s, t: (0, 0)),
            pl.BlockSpec((1, n_tiles), lambda s, t: (0, 0)),
            pl.BlockSpec((1, 1), lambda s, t: (0, 0)),
        ],
        out_shape=[
            jax.ShapeDtypeStruct((2 * T, 1), jnp.int32),
            jax.ShapeDtypeStruct((2 * T, 1), jnp.float32),
            jax.ShapeDtypeStruct((2 * T, 1), jnp.int32),
            jax.ShapeDtypeStruct((1, 16), jnp.int32),
            jax.ShapeDtypeStruct((1, n_tiles), jnp.int32),
            jax.ShapeDtypeStruct((1, 1), jnp.float32),
        ],
        scratch_shapes=[pltpu.VMEM((1, E), jnp.float32),
                        pltpu.VMEM((1, E), jnp.float32),
                        pltpu.VMEM((1, E), jnp.float32),
                        pltpu.VMEM((_TT, _TT), jnp.float32)],
    )(xb, router_w, router_b.reshape(1, E))


def _dest_body(ep_ref, rk_ref, offs_ref, dest_ref):
    ep = ep_ref[...]
    rk = rk_ref[...]
    tt = ep.shape[0]
    lane = lax.broadcasted_iota(jnp.int32, (tt, 16), 1)
    oh = (lane == ep).astype(jnp.float32)
    offs = offs_ref[...].astype(jnp.float32)
    og = jnp.sum(oh * offs, axis=1, keepdims=True)
    dest_ref[...] = rk + (og + 0.5).astype(jnp.int32)


def _dest_calc(ep, rk, offs, P):
    tile = 1024
    return pl.pallas_call(
        _dest_body,
        grid=(P // tile,),
        in_specs=[
            pl.BlockSpec((tile, 1), lambda t: (t, 0)),
            pl.BlockSpec((tile, 1), lambda t: (t, 0)),
            pl.BlockSpec((1, 16), lambda t: (0, 0)),
        ],
        out_specs=pl.BlockSpec((tile, 1), lambda t: (t, 0)),
        out_shape=jax.ShapeDtypeStruct((P, 1), jnp.int32),
    )(ep, rk, offs)


def _scatter_rows(dest, wp, x2, P, T, L, D):
    mesh = plsc.VectorSubcoreMesh(core_axis_name="c", subcore_axis_name="s")
    cpw = P // 32
    CH = 8
    NJ = cpw // CH
    NB = 5

    @functools.partial(
        pl.kernel, mesh=mesh,
        out_type=[jax.ShapeDtypeStruct((L, D), jnp.float32),
                  jax.ShapeDtypeStruct((L,), jnp.float32)],
        scratch_types=(
            [pltpu.VMEM((NJ, CH), jnp.int32),
             pltpu.VMEM((2, 128), jnp.int32),
             pltpu.VMEM((cpw,), jnp.float32)]
            + [pltpu.VMEM((CH, D), jnp.float32)] * NB
            + [pltpu.SemaphoreType.DMA] * (2 * NB + 1)),
    )
    def scat(dest_hbm, wp_hbm, x_hbm, xg_hbm, sw_hbm, didx_v, dw_v, wp_v,
             *bufs):
        rows = bufs[:NB]
        rsems = bufs[NB:2 * NB]
        ssems = bufs[2 * NB:3 * NB]
        wsem = bufs[3 * NB]
        wid = lax.axis_index("s") * 2 + lax.axis_index("c")
        base = wid * cpw
        tokbase = (wid % 16) * cpw
        for j in range(NJ):
            pltpu.sync_copy(dest_hbm.at[pl.ds(base + j * CH, CH)],
                            didx_v.at[j])
        pltpu.sync_copy(wp_hbm.at[pl.ds(base, cpw)], wp_v)
        for h in range(2):
            pltpu.sync_copy(dest_hbm.at[pl.ds(base + h * 128, 128)],
                            dw_v.at[h])
        w1 = pltpu.async_copy(wp_v.at[pl.ds(0, 128)],
                              sw_hbm.at[dw_v.at[0]], wsem)
        w2 = pltpu.async_copy(wp_v.at[pl.ds(128, 128)],
                              sw_hbm.at[dw_v.at[1]], wsem)
        rcp = [None] * NB
        scp = [None] * NB
        for j in range(NJ):
            b = j % NB
            if scp[b] is not None:
                scp[b].wait()
            rcp[b] = pltpu.async_copy(
                x_hbm.at[pl.ds(tokbase + j * CH, CH)], rows[b], rsems[b])
            if j >= 1:
                pb = (j - 1) % NB
                rcp[pb].wait()
                scp[pb] = pltpu.async_copy(
                    rows[pb], xg_hbm.at[didx_v.at[j - 1]], ssems[pb])
        lb = (NJ - 1) % NB
        rcp[lb].wait()
        scp[lb] = pltpu.async_copy(
            rows[lb], xg_hbm.at[didx_v.at[NJ - 1]], ssems[lb])
        for k in range(min(NB, NJ)):
            scp[(NJ - 1 - k) % NB].wait()
        w1.wait()
        w2.wait()

    return scat(dest, wp, x2)


def _ffn_body(te_ref, xg_ref, wu_ref, bu_ref, wd_ref, bd_ref, w_ref, y_ref):
    h = jnp.dot(xg_ref[...].astype(jnp.bfloat16), wu_ref[0],
                preferred_element_type=jnp.float32)
    h = h + bu_ref[0]
    g = 0.5 * h * (1.0 + lax.erf(h * 0.7071067811865476))
    y = jnp.dot(g.astype(jnp.bfloat16), wd_ref[0],
                preferred_element_type=jnp.float32)
    y_ref[...] = (y + bd_ref[0]) * w_ref[...]


def _ffn(te, xg, wub, bu3, wdb, bd3, sw, L, D, H, E, n_tiles):
    grid_spec = pltpu.PrefetchScalarGridSpec(
        num_scalar_prefetch=1,
        grid=(n_tiles,),
        in_specs=[
            pl.BlockSpec((_TG, D), lambda t, te: (t, 0)),
            pl.BlockSpec((1, D, H), lambda t, te: (te[t], 0, 0)),
            pl.BlockSpec((1, 1, H), lambda t, te: (te[t], 0, 0)),
            pl.BlockSpec((1, H, D), lambda t, te: (te[t], 0, 0)),
            pl.BlockSpec((1, 1, D), lambda t, te: (te[t], 0, 0)),
            pl.BlockSpec((_TG, 1), lambda t, te: (t, 0)),
        ],
        out_specs=pl.BlockSpec((_TG, D), lambda t, te: (t, 0)),
    )
    return pl.pallas_call(
        _ffn_body,
        grid_spec=grid_spec,
        out_shape=jax.ShapeDtypeStruct((L, D), jnp.float32),
    )(te, xg, wub, bu3, wdb, bd3, sw)


def _combine(dest, y, T, L, D):
    mesh = plsc.VectorSubcoreMesh(core_axis_name="c", subcore_axis_name="s")
    tpw = T // 32
    CH = 8

    NJ = tpw // CH
    UN = 32
    ni = (CH * D // 16) // UN

    NS = 3

    @functools.partial(
        pl.kernel, mesh=mesh,
        out_type=jax.ShapeDtypeStruct((T, D), jnp.float32),
        scratch_types=(
            [pltpu.VMEM((tpw,), jnp.int32), pltpu.VMEM((tpw,), jnp.int32)]
            + [pltpu.VMEM((CH, D), jnp.float32)] * (2 * NS)
            + [pltpu.SemaphoreType.DMA] * (2 * NS)),
    )
    def comb(dest_hbm, y_hbm, out_hbm, iA_v, iB_v, *bufs):
        rA = bufs[:NS]
        rB = bufs[NS:2 * NS]
        gsems = bufs[2 * NS:3 * NS]
        ssems = bufs[3 * NS:4 * NS]
        wid = lax.axis_index("s") * 2 + lax.axis_index("c")
        base = wid * tpw
        pltpu.sync_copy(dest_hbm.at[pl.ds(base, tpw)], iA_v)
        pltpu.sync_copy(dest_hbm.at[pl.ds(T + base, tpw)], iB_v)
        for q in range(tpw // 16):
            sl = pl.ds(q * 16, 16)
            iA_v[sl] = jnp.minimum(jnp.maximum(iA_v[sl], 0), L - 1)
            iB_v[sl] = jnp.minimum(jnp.maximum(iB_v[sl], 0), L - 1)
        gA = [None] * NS
        gB = [None] * NS
        scp = [None] * NS

        def do_adds(b):
            ra, rb = rA[b], rB[b]
            ngr = D // 16

            def body(i, _):
                for u in range(UN):
                    k = i * UN + u
                    r = k // ngr
                    c = (k % ngr) * 16
                    ra[r, pl.ds(c, 16)] = (ra[r, pl.ds(c, 16)]
                                           + rb[r, pl.ds(c, 16)])
                return 0

            lax.fori_loop(0, ni, body, 0)

        for j in range(NJ):
            b = j % NS
            if scp[b] is not None:
                scp[b].wait()
            gA[b] = pltpu.async_copy(
                y_hbm.at[iA_v.at[pl.ds(j * CH, CH)]], rA[b], gsems[b])
            gB[b] = pltpu.async_copy(
                y_hbm.at[iB_v.at[pl.ds(j * CH, CH)]], rB[b], gsems[b])
            if j >= 1:
                pb = (j - 1) % NS
                gA[pb].wait()
                gB[pb].wait()
                do_adds(pb)
                scp[pb] = pltpu.async_copy(
                    rA[pb], out_hbm.at[pl.ds(base + (j - 1) * CH, CH)],
                    ssems[pb])
        lb = (NJ - 1) % NS
        gA[lb].wait()
        gB[lb].wait()
        do_adds(lb)
        pltpu.sync_copy(rA[lb], out_hbm.at[pl.ds(base + (NJ - 1) * CH, CH)])
        for k in range(1, min(NS, NJ)):
            scp[(NJ - 1 - k) % NS].wait()

    return comb(dest, y)


def kernel(x, router_w, router_b, W_up, b_up, W_down, b_down):
    B, N, D = x.shape
    E, _, H = W_up.shape
    T = B * N
    P = 2 * T
    L = P + E * _TG
    n_tiles = L // _TG
    Dw = D // 2

    x2 = x.reshape(T, D)
    wub = W_up.astype(jnp.bfloat16)
    wdb = W_down.astype(jnp.bfloat16)

    ep, wp, rk, offs, te, aux = _router(
        x2, router_w, router_b, E, _TG, n_tiles)

    dest = _dest_calc(ep, rk, offs, P)

    xg, sw = _scatter_rows(dest.reshape(P), wp.reshape(P), x2, P, T, L, D)

    y = _ffn(te.reshape(n_tiles), xg, wub, b_up.reshape(E, 1, H),
             wdb, b_down.reshape(E, 1, D), sw.reshape(L, 1),
             L, D, H, E, n_tiles)

    out = _combine(dest.reshape(P), y, T, L, D)

    return out.reshape(B, N, D), aux[0, 0]
